# Initial kernel scaffold; baseline (speedup 1.0000x reference)
import jax
import jax.numpy as jnp
from jax.experimental import pallas as pl


def kernel(token_ids, weights):
    raise NotImplementedError("write your pallas kernel here")


# speedup vs baseline: 1.6851x; 1.6851x over previous
import functools

import jax
import jax.numpy as jnp
from jax import lax
from jax.experimental import pallas as pl
from jax.experimental.pallas import tpu as pltpu
from jax.experimental.pallas import tpu_sc as plsc

VOCAB = 1000000
D_MODEL = 64
BATCH = 16384
SEQ = 50
B_TOTAL = BATCH * SEQ
NUM_CORES = 2
NUM_SUBCORES = 16
NW = NUM_CORES * NUM_SUBCORES
CHUNK = 128
B_PER_W = B_TOTAL // NW
N_CHUNK = B_PER_W // CHUNK

_mesh = plsc.VectorSubcoreMesh(core_axis_name="c", subcore_axis_name="s")


@functools.partial(
    pl.kernel,
    mesh=_mesh,
    out_type=jax.ShapeDtypeStruct((B_TOTAL, D_MODEL), jnp.float32),
    scratch_types=[
        pltpu.VMEM((N_CHUNK, CHUNK), jnp.int32),
        pltpu.VMEM((CHUNK, D_MODEL), jnp.float32),
        pltpu.SemaphoreType.DMA,
    ],
    compiler_params=pltpu.CompilerParams(use_tc_tiling_on_sc=False),
)
def _gather_kernel(idx_hbm, table_hbm, out_hbm, idx_v, rows_v, sem):
    wid = lax.axis_index("s") * NUM_CORES + lax.axis_index("c")
    pltpu.sync_copy(idx_hbm.at[pl.ds(wid * N_CHUNK, N_CHUNK)], idx_v)

    def body(j, carry):
        pltpu.async_copy(table_hbm.at[idx_v.at[j]], rows_v, sem).wait()
        pltpu.sync_copy(
            rows_v, out_hbm.at[pl.ds(wid * B_PER_W + j * CHUNK, CHUNK)]
        )
        return carry

    lax.fori_loop(0, N_CHUNK, body, 0)


def kernel(token_ids, weights):
    idx = token_ids.reshape(NW * N_CHUNK, CHUNK).astype(jnp.int32)
    out = _gather_kernel(idx, weights)
    return out.reshape(BATCH, SEQ, D_MODEL)


# speedup vs baseline: 1.8748x; 1.1126x over previous
import functools

import jax
import jax.numpy as jnp
from jax import lax
from jax.experimental import pallas as pl
from jax.experimental.pallas import tpu as pltpu
from jax.experimental.pallas import tpu_sc as plsc

VOCAB = 1000000
D_MODEL = 64
BATCH = 16384
SEQ = 50
B_TOTAL = BATCH * SEQ
NUM_CORES = 2
NUM_SUBCORES = 16
NW = NUM_CORES * NUM_SUBCORES
CHUNK = 128
B_PER_W = B_TOTAL // NW
N_CHUNK = B_PER_W // CHUNK
K_GROUP = 5
G_ROWS = K_GROUP * CHUNK
N_GROUP = N_CHUNK // K_GROUP

_mesh = plsc.VectorSubcoreMesh(core_axis_name="c", subcore_axis_name="s")


@functools.partial(
    pl.kernel,
    mesh=_mesh,
    out_type=jax.ShapeDtypeStruct((B_TOTAL, D_MODEL), jnp.float32),
    scratch_types=[
        pltpu.VMEM((N_CHUNK, CHUNK), jnp.int32),
        pltpu.VMEM((G_ROWS, D_MODEL), jnp.float32),
        pltpu.VMEM((G_ROWS, D_MODEL), jnp.float32),
        pltpu.SemaphoreType.DMA,
        pltpu.SemaphoreType.DMA,
        pltpu.SemaphoreType.DMA,
        pltpu.SemaphoreType.DMA,
    ],
    compiler_params=pltpu.CompilerParams(use_tc_tiling_on_sc=False),
)
def _gather_kernel(idx_hbm, table_hbm, out_hbm, idx_v, buf0, buf1,
                   gsem0, gsem1, osem0, osem1):
    wid = lax.axis_index("s") * NUM_CORES + lax.axis_index("c")
    base_chunk = wid * N_CHUNK
    bufs = (buf0, buf1)
    gsems = (gsem0, gsem1)
    osems = (osem0, osem1)

    pltpu.sync_copy(idx_hbm.at[pl.ds(base_chunk, N_CHUNK)], idx_v)

    def fire_group(g, b):
        for k in range(K_GROUP):
            pltpu.async_copy(
                table_hbm.at[idx_v.at[g * K_GROUP + k]],
                bufs[b].at[pl.ds(k * CHUNK, CHUNK)],
                gsems[b],
            )

    def drain_group(g, b):
        for k in range(K_GROUP):
            pltpu.make_async_copy(
                table_hbm.at[idx_v.at[g * K_GROUP + k]],
                bufs[b].at[pl.ds(k * CHUNK, CHUNK)],
                gsems[b],
            ).wait()

    def fire_out(g, b):
        pltpu.async_copy(
            bufs[b],
            out_hbm.at[pl.ds((base_chunk + g * K_GROUP) * CHUNK, G_ROWS)],
            osems[b],
        )

    def wait_out(b):
        pltpu.make_async_copy(
            bufs[b], out_hbm.at[pl.ds(0, G_ROWS)], osems[b]
        ).wait()

    fire_group(0, 0)

    def outer(p, carry):
        g0 = p * 2

        @pl.when(p >= 1)
        def _():
            wait_out(1)

        fire_group(g0 + 1, 1)
        drain_group(g0, 0)
        fire_out(g0, 0)

        @pl.when(p < N_GROUP // 2 - 1)
        def _():
            wait_out(0)
            fire_group(g0 + 2, 0)

        drain_group(g0 + 1, 1)
        fire_out(g0 + 1, 1)
        return carry

    lax.fori_loop(0, N_GROUP // 2, outer, 0)
    wait_out(0)
    wait_out(1)


def kernel(token_ids, weights):
    idx = token_ids.reshape(NW * N_CHUNK, CHUNK).astype(jnp.int32)
    out = _gather_kernel(idx, weights)
    return out.reshape(BATCH, SEQ, D_MODEL)
